# Initial kernel scaffold; baseline (speedup 1.0000x reference)
import jax
import jax.numpy as jnp
from jax.experimental import pallas as pl


def kernel(inputs, xs, ys):
    raise NotImplementedError("write your pallas kernel here")


# speedup vs baseline: 2782.5435x; 2782.5435x over previous
import functools
import jax
import jax.numpy as jnp
from jax.experimental import pallas as pl
from jax.experimental.pallas import tpu as pltpu


def _next_down(v):
    bits = jax.lax.bitcast_convert_type(v, jnp.int32)
    dec = jax.lax.bitcast_convert_type(bits - 1, jnp.float32)
    neg_tiny = jnp.float32(-1e-30)
    return jnp.where(v > 0, dec, jnp.minimum(v, neg_tiny) * jnp.float32(1.0000001))


def _isotonic_block(x_ref, xs_ref, ys_ref, o_ref, *, n_bin):
    x = x_ref[...]
    xs_rows = [xs_ref[j, :] for j in range(n_bin)]
    ys_rows = [ys_ref[j, :] for j in range(n_bin)]

    u = [None] * n_bin
    u[n_bin - 1] = xs_rows[n_bin - 1]
    for j in range(n_bin - 2, -1, -1):
        u[j] = jnp.minimum(xs_rows[j], _next_down(u[j + 1]))

    acc = jnp.broadcast_to(ys_rows[0][None, :], x.shape)
    for j in range(n_bin - 1):
        w = u[j + 1] - u[j]
        s = (ys_rows[j + 1] - ys_rows[j]) / w
        t = jnp.minimum(jnp.maximum(x - u[j][None, :], 0.0), w[None, :])
        acc = acc + t * s[None, :]

    lo_mask = x <= xs_rows[0][None, :]
    hi_mask = x >= xs_rows[n_bin - 1][None, :]
    out = jnp.where(lo_mask, ys_rows[0][None, :],
                    jnp.where(hi_mask, ys_rows[n_bin - 1][None, :], acc))
    o_ref[...] = out


@jax.jit
def kernel(inputs, xs, ys):
    batch, n_unit = inputs.shape
    n_bin = xs.shape[1]
    xs_t = xs.T
    ys_t = ys.T

    bb = 1024
    grid = (batch // bb,)
    return pl.pallas_call(
        functools.partial(_isotonic_block, n_bin=n_bin),
        grid=grid,
        in_specs=[
            pl.BlockSpec((bb, n_unit), lambda i: (i, 0)),
            pl.BlockSpec((n_bin, n_unit), lambda i: (0, 0)),
            pl.BlockSpec((n_bin, n_unit), lambda i: (0, 0)),
        ],
        out_specs=pl.BlockSpec((bb, n_unit), lambda i: (i, 0)),
        out_shape=jax.ShapeDtypeStruct((batch, n_unit), jnp.float32),
    )(inputs, xs_t, ys_t)
